# Initial kernel scaffold; baseline (speedup 1.0000x reference)
import jax
import jax.numpy as jnp
from jax.experimental import pallas as pl


def kernel(h, edge_index, params):
    raise NotImplementedError("write your pallas kernel here")


# speedup vs baseline: 1.9092x; 1.9092x over previous
import functools
import math

import jax
import jax.numpy as jnp
import numpy as np
from jax import lax
from jax.experimental import pallas as pl
from jax.experimental.pallas import tpu as pltpu
from jax.experimental.pallas import tpu_sc as plsc

EPS = 1e-6
_BLADES = [0b000, 0b001, 0b010, 0b100, 0b011, 0b101, 0b110, 0b111]
_REP = np.repeat(np.arange(4), np.array([1, 3, 3, 1]))


def _sign(a, b):
    swaps = 0
    x = a >> 1
    while x:
        swaps += bin(x & b).count("1")
        x >>= 1
    return -1.0 if swaps % 2 else 1.0


def _build_cayley():
    idx = {bm: i for i, bm in enumerate(_BLADES)}
    c = np.zeros((8, 8, 8), dtype=np.float32)
    for i, bi in enumerate(_BLADES):
        for k, bk in enumerate(_BLADES):
            c[i, idx[bi ^ bk], k] = _sign(bi, bk)
    return c


_CAYLEY = _build_cayley()


def _paths():
    gs = [(0, 1), (1, 4), (4, 7), (7, 8)]
    p = np.zeros((4, 4, 4), dtype=bool)
    for gi in range(4):
        for gj in range(4):
            for gk in range(4):
                si, sj, sk = gs[gi], gs[gj], gs[gk]
                p[gi, gj, gk] = np.any(
                    _CAYLEY[si[0]:si[1], sj[0]:sj[1], sk[0]:sk[1]] != 0)
    return p


_PI, _PJ, _PK = np.nonzero(_paths())
_F = 8


def _kron_feat(block):
    return np.kron(np.eye(_F, dtype=np.float32), block.astype(np.float32))


def _static_mats():
    g = _REP
    e00 = np.zeros((8, 8), np.float32)
    e00[0, 0] = 1.0
    gq = np.zeros((8, 8), np.float32)
    gall = np.zeros((8, 8), np.float32)
    for j in range(8):
        for i in range(8):
            same = 1.0 if g[j] == g[i] else 0.0
            gall[j, i] = same
            if g[i] >= 1:
                gq[j, i] = same
    s5 = np.stack([
        _kron_feat(e00), _kron_feat(gq), _kron_feat(gall),
        _kron_feat(np.ones((8, 8), np.float32)),
        np.full((64, 64), 1.0 / 64.0, np.float32),
    ])
    r = np.kron(np.eye(64, dtype=np.float32), np.ones((1, 8), np.float32))
    c = _kron_feat(np.tile(np.eye(8, dtype=np.float32), (8, 1)))
    return jnp.asarray(s5), jnp.asarray(r), jnp.asarray(c)


_S5, _R, _C = _static_mats()


def _mvlin_mat(w):
    wrep = w[..., _REP]
    i8 = jnp.eye(8, dtype=w.dtype)
    w4 = jnp.einsum("nmi,ij->minj", wrep, i8)
    return w4.reshape(w.shape[1] * 8, w.shape[0] * 8)


def _bias_vec(b):
    return jnp.pad(b[0], ((0, 0), (0, 7))).reshape(-1)


def _grade_vec(a):
    return a[:, _REP].reshape(-1)


def _sgp_T(sgp_w):
    F = sgp_w.shape[0]
    wfull = jnp.zeros((F, 4, 4, 4), sgp_w.dtype).at[:, _PI, _PJ, _PK].set(sgp_w)
    wfull = wfull[:, _REP][:, :, _REP][:, :, :, _REP]
    weight = jnp.asarray(_CAYLEY)[None] * wfull
    blocks = weight.transpose(0, 3, 1, 2).reshape(F, 8, 64)
    eyeF = jnp.eye(F, dtype=sgp_w.dtype)
    return jnp.einsum("nkc,nm->nkmc", blocks, eyeF).reshape(F * 8, F * 64)


def _prep_block(p):
    return {
        "Wl": _mvlin_mat(p["lin"]["w"]),
        "WR": _mvlin_mat(p["sgp_right"]["w"]),
        "WL": _mvlin_mat(p["sgp_left"]["w"]),
        "T": _sgp_T(p["sgp_w"]),
        "vecs": jnp.stack([
            _bias_vec(p["lin"]["b"]),
            _grade_vec(p["silu1_a"][0]),
            _grade_vec(p["silu1_b"][0]),
            jax.nn.sigmoid(_grade_vec(p["sgp_norm_a"])),
            _bias_vec(p["sgp_left"]["b"]),
            jnp.repeat(p["ln_a"][0], 8),
            _grade_vec(p["silu2_a"][0]),
            _grade_vec(p["silu2_b"][0]),
        ]),
    }


_INV_SQRT2 = 1.0 / math.sqrt(2.0)


def _dot(a, b):
    return jax.lax.dot(a, b, precision=jax.lax.Precision.HIGHEST,
                       preferred_element_type=jnp.float32)


def _apply_block(x, Wl, WR, WL, T, vecs, s5, r, c):
    p0, gq, gall, gfeat, m = (s5[i] for i in range(5))
    bL0, a1, b1, siga_r, bL1, lna, a2, b2 = (vecs[i:i + 1] for i in range(8))
    x = _dot(x, Wl) + bL0
    norms = _dot(x, p0) + _dot(x * x, gq)
    x = jax.nn.sigmoid(a1 * norms + b1) * x
    xr = _dot(x, WR)
    q = _dot(xr * xr, gall)
    nr = jnp.sqrt(jnp.abs(q))
    xr = xr / (siga_r * (nr - 1.0) + 1.0 + EPS)
    t = _dot(xr, T)
    xrep = _dot(x, r)
    gp = _dot(xrep * t, c)
    x = (_dot(x, WL) + bL1 + gp) * _INV_SQRT2
    qf = _dot(x * x, gfeat)
    mean = _dot(jnp.sqrt(jnp.abs(qf)), m)
    x = lna * x / (mean + EPS)
    norms = _dot(x, p0) + _dot(x * x, gq)
    x = jax.nn.sigmoid(a2 * norms + b2) * x
    return x


_CHUNK = 128
_NW = 32
_RANGE = 12544
_SH_ROWS = 12672
_ZSLICE = _SH_ROWS // 16


def _sc_gather(h128, src_pad, dst_pad, epad):
    n_chunks = epad // (_NW * _CHUNK)
    per_w = n_chunks * _CHUNK
    mesh = plsc.VectorSubcoreMesh(core_axis_name="c", subcore_axis_name="s")

    @functools.partial(
        pl.kernel, mesh=mesh,
        out_type=[jax.ShapeDtypeStruct((epad, 128), jnp.float32),
                  jax.ShapeDtypeStruct((epad, 128), jnp.float32)],
        scratch_types=[
            pltpu.VMEM((_CHUNK,), jnp.int32),
            pltpu.VMEM((_CHUNK,), jnp.int32),
            pltpu.VMEM((_CHUNK, 128), jnp.float32),
            pltpu.VMEM((_CHUNK, 128), jnp.float32),
            pltpu.SemaphoreType.DMA,
            pltpu.SemaphoreType.DMA,
        ],
    )
    def k(src_hbm, dst_hbm, h_hbm, hs_out, hd_out, idx_s, idx_d, rows_s,
          rows_d, sem1, sem2):
        wid = lax.axis_index("s") * 2 + lax.axis_index("c")
        base = wid * per_w

        def body(j, carry):
            off = base + j * _CHUNK
            pltpu.sync_copy(src_hbm.at[pl.ds(off, _CHUNK)], idx_s)
            pltpu.sync_copy(dst_hbm.at[pl.ds(off, _CHUNK)], idx_d)
            cp1 = pltpu.async_copy(h_hbm.at[idx_s], rows_s, sem1)
            cp2 = pltpu.async_copy(h_hbm.at[idx_d], rows_d, sem2)
            cp1.wait()
            cp2.wait()
            pltpu.sync_copy(rows_s, hs_out.at[pl.ds(off, _CHUNK)])
            pltpu.sync_copy(rows_d, hd_out.at[pl.ds(off, _CHUNK)])
            return carry

        lax.fori_loop(0, n_chunks, body, 0)

    return k(src_pad, dst_pad, h128)


def _sc_scatter(msg128, src_scat, npad, epad):
    n_chunks = epad // (_NW * _CHUNK)
    per_w = n_chunks * _CHUNK
    n_pass = npad // _RANGE
    mesh = plsc.VectorSubcoreMesh(core_axis_name="c", subcore_axis_name="s")
    zeros = jnp.zeros((_ZSLICE, 128), jnp.float32)

    @functools.partial(
        pl.kernel, mesh=mesh,
        out_type=jax.ShapeDtypeStruct((2, npad, 128), jnp.float32),
        scratch_types=[
            pltpu.VMEM((_CHUNK,), jnp.int32),
            pltpu.VMEM((_CHUNK,), jnp.int32),
            pltpu.VMEM((_CHUNK, 128), jnp.float32),
            pltpu.VMEM_SHARED((_SH_ROWS, 128), jnp.float32),
        ],
    )
    def k(msg_hbm, idx_hbm, z_hbm, agg_out, idx_v, idx2_v, msg_v, sh):
        cid = lax.axis_index("c")
        sid = lax.axis_index("s")
        wid = sid * 2 + cid
        base = wid * per_w
        for p in range(n_pass):
            lo = p * _RANGE
            pltpu.sync_copy(z_hbm, sh.at[pl.ds(sid * _ZSLICE, _ZSLICE)])
            plsc.subcore_barrier()

            def body(j, carry):
                off = base + j * _CHUNK
                pltpu.sync_copy(idx_hbm.at[pl.ds(off, _CHUNK)], idx_v)
                pltpu.sync_copy(msg_hbm.at[pl.ds(off, _CHUNK)], msg_v)
                for i in range(_CHUNK // 16):
                    v = idx_v[pl.ds(i * 16, 16)]
                    ok = jnp.logical_and(v >= lo, v < lo + _RANGE)
                    idx2_v[pl.ds(i * 16, 16)] = jnp.where(ok, v - lo, _RANGE)
                pltpu.sync_copy(msg_v, sh.at[idx2_v], add=True)
                return carry

            lax.fori_loop(0, n_chunks, body, 0)
            plsc.subcore_barrier()
            pltpu.sync_copy(
                sh.at[pl.ds(sid * (_RANGE // 16), _RANGE // 16)],
                agg_out.at[cid, pl.ds(lo + sid * (_RANGE // 16), _RANGE // 16)])
            plsc.subcore_barrier()

    return k(msg128, src_scat, zeros)


_BE = 512


def _edge_kernel_fn(hs, hd, Wl, WR, WL, T, vecs, s5, r, c, out):
    x = hs[:, :64] - hd[:, :64]
    for b in range(2):
        x = _apply_block(x, Wl[b], WR[b], WL[b], T[b], vecs[b], s5[...],
                         r[...], c[...])
    ones = jnp.ones((x.shape[0], 1), jnp.float32)
    zeros = jnp.zeros((x.shape[0], 63), jnp.float32)
    out[...] = jnp.concatenate([x, ones, zeros], axis=1)


def _run_edge_mlp(hs, hd, blocks, epad):
    Wl = jnp.stack([b["Wl"] for b in blocks])
    WR = jnp.stack([b["WR"] for b in blocks])
    WL = jnp.stack([b["WL"] for b in blocks])
    T = jnp.stack([b["T"] for b in blocks])
    vecs = jnp.stack([b["vecs"] for b in blocks])
    full = lambda shape: pl.BlockSpec(shape, lambda i: (0,) * len(shape))
    return pl.pallas_call(
        _edge_kernel_fn,
        grid=(epad // _BE,),
        in_specs=[
            pl.BlockSpec((_BE, 128), lambda i: (i, 0)),
            pl.BlockSpec((_BE, 128), lambda i: (i, 0)),
            full((2, 64, 64)), full((2, 64, 64)), full((2, 64, 64)),
            full((2, 64, 512)), full((2, 8, 64)),
            full((5, 64, 64)), full((64, 512)), full((512, 64)),
        ],
        out_specs=pl.BlockSpec((_BE, 128), lambda i: (i, 0)),
        out_shape=jax.ShapeDtypeStruct((epad, 128), jnp.float32),
    )(hs, hd, Wl, WR, WL, T, vecs, _S5, _R, _C)


def _node_kernel_fn(h64, agg, Wl1, Wl2, WR, WL, T, vecs, s5, r, c, out):
    a = agg[0] + agg[1]
    counts = a[:, 64:65]
    aggc = a[:, :64] / jnp.maximum(counts, 1.0)
    hblk = h64[...]
    x = jnp.concatenate([hblk, aggc], axis=1)
    Wlin = [Wl1[...], Wl2[...]]
    for b in range(2):
        x = _apply_block(x, Wlin[b], WR[b], WL[b], T[b], vecs[b], s5[...],
                         r[...], c[...])
    out[...] = hblk + x


def _run_node_mlp(h64p, agg, blocks, npad):
    WR = jnp.stack([b["WR"] for b in blocks])
    WL = jnp.stack([b["WL"] for b in blocks])
    T = jnp.stack([b["T"] for b in blocks])
    vecs = jnp.stack([b["vecs"] for b in blocks])
    full = lambda shape: pl.BlockSpec(shape, lambda i: (0,) * len(shape))
    return pl.pallas_call(
        _node_kernel_fn,
        grid=(npad // _BE,),
        in_specs=[
            pl.BlockSpec((_BE, 64), lambda i: (i, 0)),
            pl.BlockSpec((2, _BE, 128), lambda i: (0, i, 0)),
            full((128, 64)), full((64, 64)),
            full((2, 64, 64)), full((2, 64, 64)),
            full((2, 64, 512)), full((2, 8, 64)),
            full((5, 64, 64)), full((64, 512)), full((512, 64)),
        ],
        out_specs=pl.BlockSpec((_BE, 64), lambda i: (i, 0)),
        out_shape=jax.ShapeDtypeStruct((npad, 64), jnp.float32),
    )(h64p, agg, blocks[0]["Wl"], blocks[1]["Wl"], WR, WL, T, vecs,
      _S5, _R, _C)


def kernel(h, edge_index, params):
    N = h.shape[0]
    E = edge_index.shape[1]
    h64 = h.reshape(N, 64).astype(jnp.float32)
    h128 = jnp.pad(h64, ((0, 0), (0, 64)))

    epad = ((E + _NW * _CHUNK - 1) // (_NW * _CHUNK)) * (_NW * _CHUNK)
    npad = _RANGE * 4

    src = edge_index[0]
    dst = edge_index[1]
    pad = epad - E
    src_g = jnp.concatenate([src, jnp.zeros((pad,), jnp.int32)])
    dst_g = jnp.concatenate([dst, jnp.zeros((pad,), jnp.int32)])
    src_s = jnp.concatenate([src, jnp.full((pad,), N, jnp.int32)])

    hs, hd = _sc_gather(h128, src_g, dst_g, epad)

    eblocks = [_prep_block(p) for p in params["edge_model"]]
    nblocks = [_prep_block(p) for p in params["node_model"]]

    msg128 = _run_edge_mlp(hs, hd, eblocks, epad)
    agg = _sc_scatter(msg128, src_s, npad, epad)

    h64p = jnp.pad(h64, ((0, npad - N), (0, 0)))
    out64 = _run_node_mlp(h64p, agg, nblocks, npad)
    return out64[:N].reshape(N, -1, 8)


# speedup vs baseline: 6.1798x; 3.2368x over previous
import functools
import math

import jax
import jax.numpy as jnp
import numpy as np
from jax import lax
from jax.experimental import pallas as pl
from jax.experimental.pallas import tpu as pltpu
from jax.experimental.pallas import tpu_sc as plsc

EPS = 1e-6
_BLADES = [0b000, 0b001, 0b010, 0b100, 0b011, 0b101, 0b110, 0b111]
_REP = np.repeat(np.arange(4), np.array([1, 3, 3, 1]))


def _sign(a, b):
    swaps = 0
    x = a >> 1
    while x:
        swaps += bin(x & b).count("1")
        x >>= 1
    return -1.0 if swaps % 2 else 1.0


def _build_cayley():
    idx = {bm: i for i, bm in enumerate(_BLADES)}
    c = np.zeros((8, 8, 8), dtype=np.float32)
    for i, bi in enumerate(_BLADES):
        for k, bk in enumerate(_BLADES):
            c[i, idx[bi ^ bk], k] = _sign(bi, bk)
    return c


_CAYLEY = _build_cayley()


def _paths():
    gs = [(0, 1), (1, 4), (4, 7), (7, 8)]
    p = np.zeros((4, 4, 4), dtype=bool)
    for gi in range(4):
        for gj in range(4):
            for gk in range(4):
                si, sj, sk = gs[gi], gs[gj], gs[gk]
                p[gi, gj, gk] = np.any(
                    _CAYLEY[si[0]:si[1], sj[0]:sj[1], sk[0]:sk[1]] != 0)
    return p


_PI, _PJ, _PK = np.nonzero(_paths())
_F = 8


def _kron_feat(block):
    return np.kron(np.eye(_F, dtype=np.float32), block.astype(np.float32))


def _static_mats():
    g = _REP
    e00 = np.zeros((8, 8), np.float32)
    e00[0, 0] = 1.0
    gq = np.zeros((8, 8), np.float32)
    gall = np.zeros((8, 8), np.float32)
    for j in range(8):
        for i in range(8):
            same = 1.0 if g[j] == g[i] else 0.0
            gall[j, i] = same
            if g[i] >= 1:
                gq[j, i] = same
    s5 = np.stack([
        _kron_feat(e00), _kron_feat(gq), _kron_feat(gall),
        _kron_feat(np.ones((8, 8), np.float32)),
        np.full((64, 64), 1.0 / 64.0, np.float32),
    ])
    r = np.kron(np.eye(64, dtype=np.float32), np.ones((1, 8), np.float32))
    c = _kron_feat(np.tile(np.eye(8, dtype=np.float32), (8, 1)))
    return jnp.asarray(s5), jnp.asarray(r), jnp.asarray(c)


_S5, _R, _C = _static_mats()


def _mvlin_mat(w):
    wrep = w[..., _REP]
    i8 = jnp.eye(8, dtype=w.dtype)
    w4 = jnp.einsum("nmi,ij->minj", wrep, i8)
    return w4.reshape(w.shape[1] * 8, w.shape[0] * 8)


def _bias_vec(b):
    return jnp.pad(b[0], ((0, 0), (0, 7))).reshape(-1)


def _grade_vec(a):
    return a[:, _REP].reshape(-1)


def _sgp_T(sgp_w):
    F = sgp_w.shape[0]
    wfull = jnp.zeros((F, 4, 4, 4), sgp_w.dtype).at[:, _PI, _PJ, _PK].set(sgp_w)
    wfull = wfull[:, _REP][:, :, _REP][:, :, :, _REP]
    weight = jnp.asarray(_CAYLEY)[None] * wfull
    blocks = weight.transpose(0, 3, 1, 2).reshape(F, 8, 64)
    eyeF = jnp.eye(F, dtype=sgp_w.dtype)
    return jnp.einsum("nkc,nm->nkmc", blocks, eyeF).reshape(F * 8, F * 64)


def _prep_block(p):
    return {
        "Wl": _mvlin_mat(p["lin"]["w"]),
        "WR": _mvlin_mat(p["sgp_right"]["w"]),
        "WL": _mvlin_mat(p["sgp_left"]["w"]),
        "T": _sgp_T(p["sgp_w"]),
        "vecs": jnp.stack([
            _bias_vec(p["lin"]["b"]),
            _grade_vec(p["silu1_a"][0]),
            _grade_vec(p["silu1_b"][0]),
            jax.nn.sigmoid(_grade_vec(p["sgp_norm_a"])),
            _bias_vec(p["sgp_left"]["b"]),
            jnp.repeat(p["ln_a"][0], 8),
            _grade_vec(p["silu2_a"][0]),
            _grade_vec(p["silu2_b"][0]),
        ]),
    }


_INV_SQRT2 = 1.0 / math.sqrt(2.0)


def _dot(a, b):
    return jax.lax.dot(a, b, precision=jax.lax.Precision.DEFAULT,
                       preferred_element_type=jnp.float32)


def _apply_block(x, Wl, WR, WL, T, vecs, s5, r, c):
    p0, gq, gall, gfeat, m = (s5[i] for i in range(5))
    bL0, a1, b1, siga_r, bL1, lna, a2, b2 = (vecs[i:i + 1] for i in range(8))
    x = _dot(x, Wl) + bL0
    norms = _dot(x, p0) + _dot(x * x, gq)
    x = jax.nn.sigmoid(a1 * norms + b1) * x
    xr = _dot(x, WR)
    q = _dot(xr * xr, gall)
    nr = jnp.sqrt(jnp.abs(q))
    xr = xr / (siga_r * (nr - 1.0) + 1.0 + EPS)
    t = _dot(xr, T)
    xrep = _dot(x, r)
    gp = _dot(xrep * t, c)
    x = (_dot(x, WL) + bL1 + gp) * _INV_SQRT2
    qf = _dot(x * x, gfeat)
    mean = _dot(jnp.sqrt(jnp.abs(qf)), m)
    x = lna * x / (mean + EPS)
    norms = _dot(x, p0) + _dot(x * x, gq)
    x = jax.nn.sigmoid(a2 * norms + b2) * x
    return x


_CHUNK = 128
_NW = 32
_RANGE = 12544
_SH_ROWS = 12672
_ZSLICE = _SH_ROWS // 16


def _sc_gather(h128, src_pad, dst_pad, epad):
    n_chunks = epad // (_NW * _CHUNK)
    per_w = n_chunks * _CHUNK
    mesh = plsc.VectorSubcoreMesh(core_axis_name="c", subcore_axis_name="s")

    @functools.partial(
        pl.kernel, mesh=mesh,
        out_type=[jax.ShapeDtypeStruct((epad, 128), jnp.float32),
                  jax.ShapeDtypeStruct((epad, 128), jnp.float32)],
        scratch_types=[
            pltpu.VMEM((_CHUNK,), jnp.int32),
            pltpu.VMEM((_CHUNK,), jnp.int32),
            pltpu.VMEM((_CHUNK, 128), jnp.float32),
            pltpu.VMEM((_CHUNK, 128), jnp.float32),
            pltpu.SemaphoreType.DMA,
            pltpu.SemaphoreType.DMA,
        ],
    )
    def k(src_hbm, dst_hbm, h_hbm, hs_out, hd_out, idx_s, idx_d, rows_s,
          rows_d, sem1, sem2):
        wid = lax.axis_index("s") * 2 + lax.axis_index("c")
        base = wid * per_w

        def body(j, carry):
            off = base + j * _CHUNK
            pltpu.sync_copy(src_hbm.at[pl.ds(off, _CHUNK)], idx_s)
            pltpu.sync_copy(dst_hbm.at[pl.ds(off, _CHUNK)], idx_d)
            cp1 = pltpu.async_copy(h_hbm.at[idx_s], rows_s, sem1)
            cp2 = pltpu.async_copy(h_hbm.at[idx_d], rows_d, sem2)
            cp1.wait()
            cp2.wait()
            pltpu.sync_copy(rows_s, hs_out.at[pl.ds(off, _CHUNK)])
            pltpu.sync_copy(rows_d, hd_out.at[pl.ds(off, _CHUNK)])
            return carry

        lax.fori_loop(0, n_chunks, body, 0)

    return k(src_pad, dst_pad, h128)


def _sc_scatter(msg128, src_scat, npad, epad):
    n_chunks = epad // (_NW * _CHUNK)
    per_w = n_chunks * _CHUNK
    n_pass = npad // _RANGE
    mesh = plsc.VectorSubcoreMesh(core_axis_name="c", subcore_axis_name="s")
    zeros = jnp.zeros((_ZSLICE, 128), jnp.float32)

    @functools.partial(
        pl.kernel, mesh=mesh,
        out_type=jax.ShapeDtypeStruct((2, npad, 128), jnp.float32),
        scratch_types=[
            pltpu.VMEM((_CHUNK,), jnp.int32),
            pltpu.VMEM((_CHUNK,), jnp.int32),
            pltpu.VMEM((_CHUNK, 128), jnp.float32),
            pltpu.VMEM_SHARED((_SH_ROWS, 128), jnp.float32),
        ],
    )
    def k(msg_hbm, idx_hbm, z_hbm, agg_out, idx_v, idx2_v, msg_v, sh):
        cid = lax.axis_index("c")
        sid = lax.axis_index("s")
        wid = sid * 2 + cid
        base = wid * per_w
        for p in range(n_pass):
            lo = p * _RANGE
            pltpu.sync_copy(z_hbm, sh.at[pl.ds(sid * _ZSLICE, _ZSLICE)])
            plsc.subcore_barrier()

            def body(j, carry):
                off = base + j * _CHUNK
                pltpu.sync_copy(idx_hbm.at[pl.ds(off, _CHUNK)], idx_v)
                pltpu.sync_copy(msg_hbm.at[pl.ds(off, _CHUNK)], msg_v)
                for i in range(_CHUNK // 16):
                    v = idx_v[pl.ds(i * 16, 16)]
                    ok = jnp.logical_and(v >= lo, v < lo + _RANGE)
                    idx2_v[pl.ds(i * 16, 16)] = jnp.where(ok, v - lo, _RANGE)
                pltpu.sync_copy(msg_v, sh.at[idx2_v], add=True)
                return carry

            lax.fori_loop(0, n_chunks, body, 0)
            plsc.subcore_barrier()
            pltpu.sync_copy(
                sh.at[pl.ds(sid * (_RANGE // 16), _RANGE // 16)],
                agg_out.at[cid, pl.ds(lo + sid * (_RANGE // 16), _RANGE // 16)])
            plsc.subcore_barrier()

    return k(msg128, src_scat, zeros)


_BE = 512


def _edge_kernel_fn(hs, hd, Wl, WR, WL, T, vecs, s5, r, c, out):
    x = hs[:, :64] - hd[:, :64]
    for b in range(2):
        x = _apply_block(x, Wl[b], WR[b], WL[b], T[b], vecs[b], s5[...],
                         r[...], c[...])
    ones = jnp.ones((x.shape[0], 1), jnp.float32)
    zeros = jnp.zeros((x.shape[0], 63), jnp.float32)
    out[...] = jnp.concatenate([x, ones, zeros], axis=1)


def _run_edge_mlp(hs, hd, blocks, epad):
    Wl = jnp.stack([b["Wl"] for b in blocks])
    WR = jnp.stack([b["WR"] for b in blocks])
    WL = jnp.stack([b["WL"] for b in blocks])
    T = jnp.stack([b["T"] for b in blocks])
    vecs = jnp.stack([b["vecs"] for b in blocks])
    full = lambda shape: pl.BlockSpec(shape, lambda i: (0,) * len(shape))
    return pl.pallas_call(
        _edge_kernel_fn,
        grid=(epad // _BE,),
        in_specs=[
            pl.BlockSpec((_BE, 128), lambda i: (i, 0)),
            pl.BlockSpec((_BE, 128), lambda i: (i, 0)),
            full((2, 64, 64)), full((2, 64, 64)), full((2, 64, 64)),
            full((2, 64, 512)), full((2, 8, 64)),
            full((5, 64, 64)), full((64, 512)), full((512, 64)),
        ],
        out_specs=pl.BlockSpec((_BE, 128), lambda i: (i, 0)),
        out_shape=jax.ShapeDtypeStruct((epad, 128), jnp.float32),
    )(hs, hd, Wl, WR, WL, T, vecs, _S5, _R, _C)


def _node_kernel_fn(h64, agg, Wl1, Wl2, WR, WL, T, vecs, s5, r, c, out):
    a = agg[0] + agg[1]
    counts = a[:, 64:65]
    aggc = a[:, :64] / jnp.maximum(counts, 1.0)
    hblk = h64[...]
    x = jnp.concatenate([hblk, aggc], axis=1)
    Wlin = [Wl1[...], Wl2[...]]
    for b in range(2):
        x = _apply_block(x, Wlin[b], WR[b], WL[b], T[b], vecs[b], s5[...],
                         r[...], c[...])
    out[...] = hblk + x


def _run_node_mlp(h64p, agg, blocks, npad):
    WR = jnp.stack([b["WR"] for b in blocks])
    WL = jnp.stack([b["WL"] for b in blocks])
    T = jnp.stack([b["T"] for b in blocks])
    vecs = jnp.stack([b["vecs"] for b in blocks])
    full = lambda shape: pl.BlockSpec(shape, lambda i: (0,) * len(shape))
    return pl.pallas_call(
        _node_kernel_fn,
        grid=(npad // _BE,),
        in_specs=[
            pl.BlockSpec((_BE, 64), lambda i: (i, 0)),
            pl.BlockSpec((2, _BE, 128), lambda i: (0, i, 0)),
            full((128, 64)), full((64, 64)),
            full((2, 64, 64)), full((2, 64, 64)),
            full((2, 64, 512)), full((2, 8, 64)),
            full((5, 64, 64)), full((64, 512)), full((512, 64)),
        ],
        out_specs=pl.BlockSpec((_BE, 64), lambda i: (i, 0)),
        out_shape=jax.ShapeDtypeStruct((npad, 64), jnp.float32),
    )(h64p, agg, blocks[0]["Wl"], blocks[1]["Wl"], WR, WL, T, vecs,
      _S5, _R, _C)


def kernel(h, edge_index, params):
    N = h.shape[0]
    E = edge_index.shape[1]
    h64 = h.reshape(N, 64).astype(jnp.float32)
    h128 = jnp.pad(h64, ((0, 0), (0, 64)))

    epad = ((E + _NW * _CHUNK - 1) // (_NW * _CHUNK)) * (_NW * _CHUNK)
    npad = _RANGE * 4

    src = edge_index[0]
    dst = edge_index[1]
    pad = epad - E
    src_g = jnp.concatenate([src, jnp.zeros((pad,), jnp.int32)])
    dst_g = jnp.concatenate([dst, jnp.zeros((pad,), jnp.int32)])
    src_s = jnp.concatenate([src, jnp.full((pad,), N, jnp.int32)])

    hs, hd = _sc_gather(h128, src_g, dst_g, epad)

    eblocks = [_prep_block(p) for p in params["edge_model"]]
    nblocks = [_prep_block(p) for p in params["node_model"]]

    msg128 = _run_edge_mlp(hs, hd, eblocks, epad)
    agg = _sc_scatter(msg128, src_s, npad, epad)

    h64p = jnp.pad(h64, ((0, npad - N), (0, 0)))
    out64 = _run_node_mlp(h64p, agg, nblocks, npad)
    return out64[:N].reshape(N, -1, 8)
